# speedup vs baseline: 27.5213x; 27.5213x over previous
import jax
import jax.numpy as jnp
from jax import lax
from jax.experimental import pallas as pl
from jax.experimental.pallas import tpu as pltpu

_PH, _PW = 7, 7
_SCALE = 0.0625


def _roi_pool_body(bb, yy, xx, hh, ww, feat, out, rm):
    r = pl.program_id(0)
    b = bb[r]
    y1 = yy[r]
    x1 = xx[r]
    h = hh[r]
    w = ww[r]
    H = feat.shape[1]
    W = feat.shape[2]

    for i in range(_PH):
        s = jnp.minimum(y1 + (i * h) // _PH, H - 1)
        e = jnp.minimum(y1 + ((i + 1) * h + (_PH - 1)) // _PH, H)
        acc = feat[b, s, :, :]
        acc = lax.fori_loop(
            s + 1, e, lambda y, a: jnp.maximum(a, feat[b, y, :, :]), acc
        )
        rm[i] = acc

    for j in range(_PW):
        s = jnp.minimum(x1 + (j * w) // _PW, W - 1)
        e = jnp.minimum(x1 + ((j + 1) * w + (_PW - 1)) // _PW, W)
        acc = rm[:, s, :]
        acc = lax.fori_loop(
            s + 1, e, lambda x, a: jnp.maximum(a, rm[:, x, :]), acc
        )
        out[0, j] = acc


def kernel(features, rois):
    B, C, H, W = features.shape
    R = rois.shape[0]
    r32 = rois.astype(jnp.int32)
    coords = jnp.floor(r32[:, 1:].astype(jnp.float32) * _SCALE).astype(jnp.int32)
    b = r32[:, 0]
    x1, y1, x2, y2 = coords[:, 0], coords[:, 1], coords[:, 2], coords[:, 3]
    h = y2 - y1 + 1
    w = x2 - x1 + 1

    feat = jnp.transpose(features, (0, 2, 3, 1))

    smem = pl.BlockSpec(memory_space=pltpu.SMEM)
    out = pl.pallas_call(
        _roi_pool_body,
        grid=(R,),
        in_specs=[
            smem, smem, smem, smem, smem,
            pl.BlockSpec((B, H, W, C), lambda r: (0, 0, 0, 0)),
        ],
        out_specs=pl.BlockSpec((1, _PW, _PH, C), lambda r: (r, 0, 0, 0)),
        out_shape=jax.ShapeDtypeStruct((R, _PW, _PH, C), jnp.float32),
        scratch_shapes=[pltpu.VMEM((_PH, W, C), jnp.float32)],
        compiler_params=pltpu.CompilerParams(
            dimension_semantics=("parallel",),
            vmem_limit_bytes=48 * 1024 * 1024,
        ),
    )(b, y1, x1, h, w, feat)

    return jnp.transpose(out, (0, 3, 2, 1))


# speedup vs baseline: 28.7339x; 1.0441x over previous
import jax
import jax.numpy as jnp
from jax import lax
from jax.experimental import pallas as pl
from jax.experimental.pallas import tpu as pltpu

_PH, _PW = 7, 7
_SCALE = 0.0625


def _roi_pool_body(bb, yy, xx, hh, ww, feat, out, rm):
    r = pl.program_id(0)
    b = bb[r]
    y1 = yy[r]
    x1 = xx[r]
    h = hh[r]
    w = ww[r]
    H = feat.shape[1]
    W = feat.shape[2]

    for i in range(_PH):
        s = jnp.minimum(y1 + (i * h) // _PH, H - 1)
        e = jnp.minimum(y1 + ((i + 1) * h + (_PH - 1)) // _PH, H)
        last = e - 1
        acc = feat[b, s, :, :]
        for k in range(1, 9):
            acc = jnp.maximum(acc, feat[b, jnp.minimum(s + k, last), :, :])
        rm[i] = acc

    for j in range(_PW):
        s = jnp.minimum(x1 + (j * w) // _PW, W - 1)
        e = jnp.minimum(x1 + ((j + 1) * w + (_PW - 1)) // _PW, W)
        last = e - 1
        acc = rm[:, s, :]
        for k in range(1, 9):
            acc = jnp.maximum(acc, rm[:, jnp.minimum(s + k, last), :])
        out[0, j] = acc


def kernel(features, rois):
    B, C, H, W = features.shape
    R = rois.shape[0]
    r32 = rois.astype(jnp.int32)
    coords = jnp.floor(r32[:, 1:].astype(jnp.float32) * _SCALE).astype(jnp.int32)
    b = r32[:, 0]
    x1, y1, x2, y2 = coords[:, 0], coords[:, 1], coords[:, 2], coords[:, 3]
    h = y2 - y1 + 1
    w = x2 - x1 + 1

    feat = jnp.transpose(features, (0, 2, 3, 1))

    smem = pl.BlockSpec(memory_space=pltpu.SMEM)
    out = pl.pallas_call(
        _roi_pool_body,
        grid=(R,),
        in_specs=[
            smem, smem, smem, smem, smem,
            pl.BlockSpec((B, H, W, C), lambda r: (0, 0, 0, 0)),
        ],
        out_specs=pl.BlockSpec((1, _PW, _PH, C), lambda r: (r, 0, 0, 0)),
        out_shape=jax.ShapeDtypeStruct((R, _PW, _PH, C), jnp.float32),
        scratch_shapes=[pltpu.VMEM((_PH, W, C), jnp.float32)],
        compiler_params=pltpu.CompilerParams(
            dimension_semantics=("parallel",),
            vmem_limit_bytes=48 * 1024 * 1024,
        ),
    )(b, y1, x1, h, w, feat)

    return jnp.transpose(out, (0, 3, 2, 1))


# speedup vs baseline: 33.5982x; 1.1693x over previous
import jax
import jax.numpy as jnp
from jax.experimental import pallas as pl
from jax.experimental.pallas import tpu as pltpu

_PH, _PW = 7, 7
_SCALE = 0.0625
_NLVL = 4


def _roi_pool_body(rkb, rlo, rhi, cix, feat, out, t, rm, sem):
    r = pl.program_id(0)
    B = feat.shape[0]
    H = feat.shape[1]
    W = feat.shape[2]
    C = feat.shape[3]

    @pl.when(r == 0)
    def _init():
        cp = pltpu.make_async_copy(feat, t.at[0:B], sem)
        cp.start()
        cp.wait()
        for l in range(1, _NLVL):
            d = 1 << (l - 1)
            for b_ in range(B):
                src = (l - 1) * B + b_
                dst = l * B + b_
                t[dst, 0:H - d] = jnp.maximum(t[src, 0:H - d], t[src, d:H])
                t[dst, H - d:H] = jnp.maximum(
                    t[src, H - d:H],
                    jnp.broadcast_to(t[src, H - 1:H], (d, W, C)),
                )

    base7 = r * _PH
    for i in range(_PH):
        kb = rkb[base7 + i]
        rm[i] = jnp.maximum(t[kb, rlo[base7 + i]], t[kb, rhi[base7 + i]])

    base9 = r * (_PW * 9)
    for j in range(_PW):
        o = base9 + j * 9
        acc = rm[:, cix[o], :]
        for k in range(1, 9):
            acc = jnp.maximum(acc, rm[:, cix[o + k], :])
        out[0, j] = acc


def kernel(features, rois):
    B, C, H, W = features.shape
    R = rois.shape[0]
    r32 = rois.astype(jnp.int32)
    coords = jnp.floor(r32[:, 1:].astype(jnp.float32) * _SCALE).astype(jnp.int32)
    b = r32[:, 0]
    x1, y1, x2, y2 = coords[:, 0], coords[:, 1], coords[:, 2], coords[:, 3]
    h = y2 - y1 + 1
    w = x2 - x1 + 1

    i7 = jnp.arange(_PH, dtype=jnp.int32)
    sh = jnp.minimum(y1[:, None] + (i7 * h[:, None]) // _PH, H - 1)
    eh = jnp.minimum(y1[:, None] + ((i7 + 1) * h[:, None] + _PH - 1) // _PH, H)
    nh = eh - sh
    kh = ((nh > 1).astype(jnp.int32) + (nh > 3).astype(jnp.int32)
          + (nh > 7).astype(jnp.int32))
    rkb = (kh * B + b[:, None]).reshape(-1)
    rlo = sh.reshape(-1)
    rhi = (eh - jnp.left_shift(jnp.int32(1), kh)).reshape(-1)

    sw = jnp.minimum(x1[:, None] + (i7 * w[:, None]) // _PW, W - 1)
    ew = jnp.minimum(x1[:, None] + ((i7 + 1) * w[:, None] + _PW - 1) // _PW, W)
    k9 = jnp.arange(9, dtype=jnp.int32)
    cix = jnp.minimum(sw[:, :, None] + k9, (ew - 1)[:, :, None]).reshape(-1)

    feat = jnp.transpose(features, (0, 2, 3, 1))

    smem = pl.BlockSpec(memory_space=pltpu.SMEM)
    out = pl.pallas_call(
        _roi_pool_body,
        grid=(R,),
        in_specs=[
            smem, smem, smem, smem,
            pl.BlockSpec(memory_space=pl.ANY),
        ],
        out_specs=pl.BlockSpec((1, _PW, _PH, C), lambda r: (r, 0, 0, 0)),
        out_shape=jax.ShapeDtypeStruct((R, _PW, _PH, C), jnp.float32),
        scratch_shapes=[
            pltpu.VMEM((_NLVL * B, H, W, C), jnp.float32),
            pltpu.VMEM((_PH, W, C), jnp.float32),
            pltpu.SemaphoreType.DMA,
        ],
        compiler_params=pltpu.CompilerParams(
            dimension_semantics=("arbitrary",),
            vmem_limit_bytes=50 * 1024 * 1024,
        ),
    )(rkb, rlo, rhi, cix, feat)

    return jnp.transpose(out, (0, 3, 2, 1))


# speedup vs baseline: 48.0494x; 1.4301x over previous
import jax
import jax.numpy as jnp
from jax.experimental import pallas as pl
from jax.experimental.pallas import tpu as pltpu

_PH, _PW = 7, 7
_SCALE = 0.0625
_NLVL = 4
_G = 8


def _roi_pool_body(rkb, rlo, rhi, cix, feat, out, t, rm, sem):
    r = pl.program_id(0)
    B = feat.shape[0]
    H = feat.shape[1]
    W = feat.shape[2]
    C = feat.shape[3]

    @pl.when(r == 0)
    def _init():
        cp = pltpu.make_async_copy(feat, t.at[0:B], sem)
        cp.start()
        cp.wait()
        for l in range(1, _NLVL):
            d = 1 << (l - 1)
            for b_ in range(B):
                src = (l - 1) * B + b_
                dst = l * B + b_
                t[dst, 0:H - d] = jnp.maximum(t[src, 0:H - d], t[src, d:H])
                t[dst, H - d:H] = jnp.maximum(
                    t[src, H - d:H],
                    jnp.broadcast_to(t[src, H - 1:H], (d, W, C)),
                )

    for g in range(_G):
        base7 = (r * _G + g) * _PH
        for i in range(_PH):
            kb = rkb[base7 + i]
            rm[g, i] = jnp.maximum(t[kb, rlo[base7 + i]], t[kb, rhi[base7 + i]])

    for g in range(_G):
        base9 = (r * _G + g) * (_PW * 9)
        for j in range(_PW):
            o = base9 + j * 9
            acc = rm[g, :, cix[o], :]
            for k in range(1, 9):
                acc = jnp.maximum(acc, rm[g, :, cix[o + k], :])
            out[g, j] = acc


def kernel(features, rois):
    B, C, H, W = features.shape
    R = rois.shape[0]
    r32 = rois.astype(jnp.int32)
    coords = jnp.floor(r32[:, 1:].astype(jnp.float32) * _SCALE).astype(jnp.int32)
    b = r32[:, 0]
    x1, y1, x2, y2 = coords[:, 0], coords[:, 1], coords[:, 2], coords[:, 3]
    h = y2 - y1 + 1
    w = x2 - x1 + 1

    i7 = jnp.arange(_PH, dtype=jnp.int32)
    sh = jnp.minimum(y1[:, None] + (i7 * h[:, None]) // _PH, H - 1)
    eh = jnp.minimum(y1[:, None] + ((i7 + 1) * h[:, None] + _PH - 1) // _PH, H)
    nh = eh - sh
    kh = ((nh > 1).astype(jnp.int32) + (nh > 3).astype(jnp.int32)
          + (nh > 7).astype(jnp.int32))
    rkb = (kh * B + b[:, None]).reshape(-1)
    rlo = sh.reshape(-1)
    rhi = (eh - jnp.left_shift(jnp.int32(1), kh)).reshape(-1)

    sw = jnp.minimum(x1[:, None] + (i7 * w[:, None]) // _PW, W - 1)
    ew = jnp.minimum(x1[:, None] + ((i7 + 1) * w[:, None] + _PW - 1) // _PW, W)
    k9 = jnp.arange(9, dtype=jnp.int32)
    cix = jnp.minimum(sw[:, :, None] + k9, (ew - 1)[:, :, None]).reshape(-1)

    feat = jnp.transpose(features, (0, 2, 3, 1))

    smem = pl.BlockSpec(memory_space=pltpu.SMEM)
    out = pl.pallas_call(
        _roi_pool_body,
        grid=(R // _G,),
        in_specs=[
            smem, smem, smem, smem,
            pl.BlockSpec(memory_space=pl.ANY),
        ],
        out_specs=pl.BlockSpec((_G, _PW, _PH, C), lambda r: (r, 0, 0, 0)),
        out_shape=jax.ShapeDtypeStruct((R, _PW, _PH, C), jnp.float32),
        scratch_shapes=[
            pltpu.VMEM((_NLVL * B, H, W, C), jnp.float32),
            pltpu.VMEM((_G, _PH, W, C), jnp.float32),
            pltpu.SemaphoreType.DMA,
        ],
        compiler_params=pltpu.CompilerParams(
            dimension_semantics=("arbitrary",),
            vmem_limit_bytes=50 * 1024 * 1024,
        ),
    )(rkb, rlo, rhi, cix, feat)

    return jnp.transpose(out, (0, 3, 2, 1))
